# baseline (speedup 1.0000x reference)
import functools

import jax
import jax.numpy as jnp
from jax import lax
from jax.experimental import pallas as pl
from jax.experimental.pallas import tpu as pltpu
from jax.experimental.pallas import tpu_sc as plsc

NC = 2
NS = 16
L = 16
CHUNK = 128


@functools.lru_cache(maxsize=None)
def _build(B, D):
    assert D == L
    NW = NC * NS
    BPW = B // NW
    assert BPW % CHUNK == 0
    NCH = BPW // CHUNK

    mesh = plsc.VectorSubcoreMesh(
        core_axis_name="c", subcore_axis_name="s",
        num_cores=NC, num_subcores=NS)

    @functools.partial(
        pl.kernel,
        out_type=jax.ShapeDtypeStruct((B,), jnp.float32),
        mesh=mesh,
        compiler_params=pltpu.CompilerParams(
            needs_layout_passes=False, use_tc_tiling_on_sc=False),
        scratch_types=[
            pltpu.VMEM((BPW,), jnp.int32),
            pltpu.VMEM((BPW,), jnp.int32),
            pltpu.VMEM((BPW,), jnp.int32),
            pltpu.VMEM((BPW,), jnp.int32),
            pltpu.VMEM((CHUNK, 128), jnp.float32),
            pltpu.VMEM((CHUNK, 128), jnp.float32),
            pltpu.VMEM((CHUNK, 128), jnp.float32),
            pltpu.VMEM((CHUNK, 128), jnp.float32),
            pltpu.VMEM((BPW,), jnp.float32),
            pltpu.VMEM((BPW,), jnp.float32),
            pltpu.SemaphoreType.DMA,
            pltpu.SemaphoreType.DMA,
        ],
    )
    def hybrid_kernel(u_hbm, i_hbm, mod_u_hbm, mod_i_hbm, mem_u_hbm,
                      mem_i_hbm, alpha_hbm, out_hbm,
                      u_v, i_v, u8_v, i8_v, mu_v, mi_v, ku_v, ki_v,
                      a_v, o_v, sem, sem_a):
        wid = lax.axis_index("s") * NC + lax.axis_index("c")
        base = wid * BPW

        pltpu.sync_copy(u_hbm.at[pl.ds(base, BPW)], u_v)
        pltpu.sync_copy(i_hbm.at[pl.ds(base, BPW)], i_v)

        iota = lax.iota(jnp.int32, L)

        def shift_body(b, carry):
            s = pl.ds(b * L, L)
            u8_v[s] = lax.shift_right_logical(u_v[s], 3)
            i8_v[s] = lax.shift_right_logical(i_v[s], 3)
            return carry

        lax.fori_loop(0, BPW // L, shift_body, 0, unroll=False)

        a_copies = []
        for j in range(NCH):
            s = pl.ds(j * CHUNK, CHUNK)
            a_copies.append(
                pltpu.async_copy(alpha_hbm.at[u_v.at[s]], a_v.at[s], sem_a))
        for c in a_copies:
            c.wait()

        for j in range(NCH):
            s = pl.ds(j * CHUNK, CHUNK)
            copies = [
                pltpu.async_copy(mod_u_hbm.at[u8_v.at[s]], mu_v, sem),
                pltpu.async_copy(mod_i_hbm.at[i8_v.at[s]], mi_v, sem),
                pltpu.async_copy(mem_u_hbm.at[u8_v.at[s]], ku_v, sem),
                pltpu.async_copy(mem_i_hbm.at[i8_v.at[s]], ki_v, sem),
            ]
            for c in copies:
                c.wait()

            def blk_body(b, carry):
                pos = j * CHUNK + b * L
                rows = b * L + iota
                ucol = (u_v[pl.ds(pos, L)] & 7) * L
                icol = (i_v[pl.ds(pos, L)] & 7) * L
                acc1 = jnp.zeros((L,), jnp.float32)
                acc2 = jnp.zeros((L,), jnp.float32)
                for d in range(D):
                    acc1 = acc1 + (plsc.load_gather(mu_v, [rows, ucol + d])
                                   * plsc.load_gather(mi_v, [rows, icol + d]))
                    acc2 = acc2 + (plsc.load_gather(ku_v, [rows, ucol + d])
                                   * plsc.load_gather(ki_v, [rows, icol + d]))
                a = a_v[pl.ds(pos, L)]
                o_v[pl.ds(pos, L)] = a * acc1 + (1.0 - a) * acc2
                return carry

            lax.fori_loop(0, CHUNK // L, blk_body, 0, unroll=False)

        pltpu.sync_copy(o_v, out_hbm.at[pl.ds(base, BPW)])

    return hybrid_kernel


def kernel(user_indices, item_indices, mod_user_emb, mod_item_emb,
           mem_user_emb, mem_item_emb, alpha_table):
    B = user_indices.shape[0]
    N, D = mod_user_emb.shape
    g = (N * D) // 128
    return _build(B, D)(
        user_indices, item_indices,
        mod_user_emb.reshape(g, 128), mod_item_emb.reshape(g, 128),
        mem_user_emb.reshape(g, 128), mem_item_emb.reshape(g, 128),
        alpha_table.reshape(-1))


# speedup vs baseline: 1.0009x; 1.0009x over previous
import functools

import jax
import jax.numpy as jnp
from jax import lax
from jax.experimental import pallas as pl
from jax.experimental.pallas import tpu as pltpu
from jax.experimental.pallas import tpu_sc as plsc

NC = 2
NS = 16
L = 16
CHUNK = 128


@functools.lru_cache(maxsize=None)
def _build(B, D):
    assert D == L
    NW = NC * NS
    BPW = B // NW
    assert BPW % CHUNK == 0
    NCH = BPW // CHUNK

    mesh = plsc.VectorSubcoreMesh(
        core_axis_name="c", subcore_axis_name="s",
        num_cores=NC, num_subcores=NS)

    @functools.partial(
        pl.kernel,
        out_type=jax.ShapeDtypeStruct((B,), jnp.float32),
        mesh=mesh,
        compiler_params=pltpu.CompilerParams(needs_layout_passes=False),
        scratch_types=[
            pltpu.VMEM((BPW,), jnp.int32),
            pltpu.VMEM((BPW,), jnp.int32),
            pltpu.VMEM((BPW,), jnp.int32),
            pltpu.VMEM((BPW,), jnp.int32),
            pltpu.VMEM((CHUNK, 128), jnp.float32),
            pltpu.VMEM((CHUNK, 128), jnp.float32),
            pltpu.VMEM((CHUNK, 128), jnp.float32),
            pltpu.VMEM((CHUNK, 128), jnp.float32),
            pltpu.VMEM((BPW,), jnp.float32),
            pltpu.VMEM((BPW,), jnp.float32),
            pltpu.SemaphoreType.DMA,
            pltpu.SemaphoreType.DMA,
        ],
    )
    def hybrid_kernel(u_hbm, i_hbm, mod_u_hbm, mod_i_hbm, mem_u_hbm,
                      mem_i_hbm, alpha_hbm, out_hbm,
                      u_v, i_v, u8_v, i8_v, mu_v, mi_v, ku_v, ki_v,
                      a_v, o_v, sem, sem_a):
        wid = lax.axis_index("s") * NC + lax.axis_index("c")
        base = wid * BPW

        pltpu.sync_copy(u_hbm.at[pl.ds(base, BPW)], u_v)
        pltpu.sync_copy(i_hbm.at[pl.ds(base, BPW)], i_v)

        iota = lax.iota(jnp.int32, L)

        def shift_body(b, carry):
            s = pl.ds(b * L, L)
            u8_v[s] = lax.shift_right_logical(u_v[s], 3)
            i8_v[s] = lax.shift_right_logical(i_v[s], 3)
            return carry

        lax.fori_loop(0, BPW // L, shift_body, 0, unroll=False)

        a_copies = []
        for j in range(NCH):
            s = pl.ds(j * CHUNK, CHUNK)
            a_copies.append(
                pltpu.async_copy(alpha_hbm.at[u_v.at[s]], a_v.at[s], sem_a))
        for c in a_copies:
            c.wait()

        for j in range(NCH):
            s = pl.ds(j * CHUNK, CHUNK)
            copies = [
                pltpu.async_copy(mod_u_hbm.at[u8_v.at[s]], mu_v, sem),
                pltpu.async_copy(mod_i_hbm.at[i8_v.at[s]], mi_v, sem),
                pltpu.async_copy(mem_u_hbm.at[u8_v.at[s]], ku_v, sem),
                pltpu.async_copy(mem_i_hbm.at[i8_v.at[s]], ki_v, sem),
            ]
            for c in copies:
                c.wait()

            def blk_body(b, carry):
                pos = j * CHUNK + b * L
                rows = b * L + iota
                ucol = (u_v[pl.ds(pos, L)] & 7) * L
                icol = (i_v[pl.ds(pos, L)] & 7) * L
                acc1 = jnp.zeros((L,), jnp.float32)
                acc2 = jnp.zeros((L,), jnp.float32)
                for d in range(D):
                    acc1 = acc1 + (plsc.load_gather(mu_v, [rows, ucol + d])
                                   * plsc.load_gather(mi_v, [rows, icol + d]))
                    acc2 = acc2 + (plsc.load_gather(ku_v, [rows, ucol + d])
                                   * plsc.load_gather(ki_v, [rows, icol + d]))
                a = a_v[pl.ds(pos, L)]
                o_v[pl.ds(pos, L)] = a * acc1 + (1.0 - a) * acc2
                return carry

            lax.fori_loop(0, CHUNK // L, blk_body, 0, unroll=False)

        pltpu.sync_copy(o_v, out_hbm.at[pl.ds(base, BPW)])

    return hybrid_kernel


def kernel(user_indices, item_indices, mod_user_emb, mod_item_emb,
           mem_user_emb, mem_item_emb, alpha_table):
    B = user_indices.shape[0]
    N, D = mod_user_emb.shape
    g = (N * D) // 128
    return _build(B, D)(
        user_indices, item_indices,
        mod_user_emb.reshape(g, 128), mod_item_emb.reshape(g, 128),
        mem_user_emb.reshape(g, 128), mem_item_emb.reshape(g, 128),
        alpha_table.reshape(-1))
